# baseline (speedup 1.0000x reference)
import functools

import jax
import jax.numpy as jnp
from jax import lax
from jax.experimental import pallas as pl
from jax.experimental.pallas import tpu as pltpu
from jax.experimental.pallas import tpu_sc as plsc

VOCAB = 1000000
EMBED_DIM = 32

NC = 2
NS = 16
NW = NC * NS

B = 4096
T = 200
CHUNK = 128
JB = CHUNK // 128
BLK = B // CHUNK
N_CHUNKS = (T * BLK) // NW
PER_W = N_CHUNKS * CHUNK
IDXROWS = PER_W // B + 1
NBUF = 3


def _make_kernel():
  mesh = plsc.VectorSubcoreMesh(
      core_axis_name="c", subcore_axis_name="s", num_cores=NC,
      num_subcores=NS)

  @functools.partial(
      pl.kernel,
      out_type=jax.ShapeDtypeStruct((T, 4, B // 128, 8, 128),
                                    jnp.float32),
      mesh=mesh,
      scratch_types=[
          pltpu.VMEM((IDXROWS, B), jnp.int32),
          pltpu.VMEM((NBUF, CHUNK, EMBED_DIM), jnp.float32),
          pltpu.VMEM((NBUF, 4, JB, 8, 128), jnp.float32),
          pltpu.SemaphoreType.DMA((NBUF,)),
          pltpu.SemaphoreType.DMA((NBUF,)),
      ],
      compiler_params=pltpu.CompilerParams(
          use_tc_tiling_on_sc=False, needs_layout_passes=False),
  )
  def gather_kernel(idx_hbm, table_hbm, out_hbm, idx_all, rows_v, z_v,
                    sem_g, sem_o):
    wid = lax.axis_index("s") * NC + lax.axis_index("c")
    t0 = (wid * PER_W) // B
    off0 = wid * PER_W - t0 * B
    pltpu.sync_copy(idx_hbm.at[pl.ds(t0, IDXROWS)], idx_all)
    iot = lax.iota(jnp.int32, 16)

    def gather(c, b):
      p = off0 + c * CHUNK
      return pltpu.make_async_copy(
          table_hbm.at[idx_all.at[p // B, pl.ds(p % B, CHUNK)]],
          rows_v.at[b], sem_g.at[b])

    def transpose(b):
      rows = rows_v.at[b]
      for kt in range(4):
        for j in range(JB):
          for ks in range(8):
            col = jnp.full((16,), kt * 8 + ks, jnp.int32)
            for blg in range(8):
              row = iot + (j * 128 + blg * 16)
              vals = plsc.load_gather(rows, [row, col])
              z_v[b, kt, j, ks, pl.ds(blg * 16, 16)] = vals

    def writeback(c, b):
      g = wid * N_CHUNKS + c
      t = g // BLK
      bt0 = (g % BLK) * JB
      return pltpu.make_async_copy(
          z_v.at[b], out_hbm.at[t, :, pl.ds(bt0, JB)], sem_o.at[b])

    for c in range(NBUF):
      gather(c, c).start()
    for c in range(NBUF):
      gather(c, c).wait()
      transpose(c)
      writeback(c, c).start()
      gather(c + NBUF, c).start()

    @pl.loop(NBUF, NBUF * ((N_CHUNKS - NBUF - 1) // NBUF), step=NBUF)
    def _grp(g):
      for i in range(NBUF):
        c = g + i
        b = i
        writeback(c - NBUF, b).wait()
        gather(c, b).wait()
        transpose(b)
        writeback(c, b).start()
        gather(c + NBUF, b).start()

    for c in range(NBUF * ((N_CHUNKS - NBUF - 1) // NBUF), N_CHUNKS):
      b = c % NBUF
      writeback(c - NBUF, b).wait()
      gather(c, b).wait()
      transpose(b)
      writeback(c, b).start()
      if c + NBUF < N_CHUNKS:
        gather(c + NBUF, b).start()
    for c in range(N_CHUNKS - NBUF, N_CHUNKS):
      writeback(c, c % NBUF).wait()

  return gather_kernel


_gather = _make_kernel()


@jax.jit
def kernel(token_indices, embedding_table):
  idx_t = token_indices.T.astype(jnp.int32)
  z = _gather(idx_t, embedding_table)
  return z.transpose(2, 4, 0, 1, 3).reshape(B, T, EMBED_DIM)


# speedup vs baseline: 1.2206x; 1.2206x over previous
import functools

import jax
import jax.numpy as jnp
from jax import lax
from jax.experimental import pallas as pl
from jax.experimental.pallas import tpu as pltpu
from jax.experimental.pallas import tpu_sc as plsc

VOCAB = 1000000
EMBED_DIM = 32

NC = 2
NS = 16
NW = NC * NS

B = 4096
T = 200
RT = T // 8
CT = B // 128
N_PAIRS = (RT * CT) // NW
NBUF = 4


def _make_kernel():
  mesh = plsc.VectorSubcoreMesh(
      core_axis_name="c", subcore_axis_name="s", num_cores=NC,
      num_subcores=NS)

  @functools.partial(
      pl.kernel,
      out_type=jax.ShapeDtypeStruct((B, T, EMBED_DIM), jnp.float32),
      mesh=mesh,
      scratch_types=[
          pltpu.VMEM((2, 8, 128), jnp.int32),
          pltpu.VMEM((NBUF, 128, EMBED_DIM), jnp.float32),
          pltpu.SemaphoreType.DMA((2,)),
          pltpu.SemaphoreType.DMA((NBUF,)),
          pltpu.SemaphoreType.DMA((NBUF,)),
      ],
      compiler_params=pltpu.CompilerParams(use_tc_tiling_on_sc=False),
  )
  def gather_kernel(idx_hbm, table_hbm, out_hbm, xbuf, rows_v, sem_i,
                    sem_g, sem_o):
    wid = lax.axis_index("s") * NC + lax.axis_index("c")
    p0 = wid * N_PAIRS

    def idx_load(q, xs):
      p = p0 + q
      return pltpu.make_async_copy(
          idx_hbm.at[p // CT, p % CT], xbuf.at[xs], sem_i.at[xs])

    def gather(q, rs, xs, b):
      return pltpu.make_async_copy(
          table_hbm.at[xbuf.at[xs, rs]], rows_v.at[b], sem_g.at[b])

    def writeback(q, rs, b):
      p = p0 + q
      t = (p // CT) * 8 + rs
      b0 = (p % CT) * 128
      return pltpu.make_async_copy(
          rows_v.at[b], out_hbm.at[pl.ds(b0, 128), t], sem_o.at[b])

    def do_pair(q, xs, first, last):
      if not last:
        idx_load(q + 1, (xs + 1) % 2).start()
      for rs in range(8):
        b = rs % NBUF
        if rs == 5 and not last:
          idx_load(q + 1, (xs + 1) % 2).wait()
        if not (first and rs == 0):
          wq, wrs = (q, rs - 1) if rs >= 1 else (q - 1, 7)
          writeback(wq, wrs, (b - 1) % NBUF).wait()
        if not last or rs <= 4:
          nq, nrs = (q, rs + 3) if rs < 5 else (q + 1, rs - 5)
          nxs = xs if rs < 5 else (xs + 1) % 2
          gather(nq, nrs, nxs, (b + 3) % NBUF).start()
        gather(q, rs, xs, b).wait()
        writeback(q, rs, b).start()

    idx_load(0, 0).start()
    idx_load(0, 0).wait()
    for rs in range(3):
      gather(0, rs, 0, rs).start()
    do_pair(0, 0, True, False)

    @pl.loop(1, 23, step=2)
    def _grp(g):
      do_pair(g, 1, False, False)
      do_pair(g + 1, 0, False, False)

    do_pair(23, 1, False, False)
    do_pair(24, 0, False, True)
    writeback(24, 7, 3).wait()

  return gather_kernel


_gather = _make_kernel()


@jax.jit
def kernel(token_indices, embedding_table):
  idx4 = (token_indices.T.astype(jnp.int32)
          .reshape(RT, 8, CT, 128).transpose(0, 2, 1, 3))
  return _gather(idx4, embedding_table)
